# baseline (speedup 1.0000x reference)
import jax
import jax.numpy as jnp
import numpy as np
from jax import lax
from jax.experimental import pallas as pl
from jax.experimental.pallas import tpu as pltpu
from jax.experimental.pallas import tpu_sc as plsc

_B = 128
_N = 8192
_K = 2457
_ALPHA = 2.0
_NSTAGE = 3
_INT_MIN = np.int32(-(2 ** 31))

_NC = 2
_NS = 16
_NW = _NC * _NS
_RPW = _B // _NW
_VL = 16
_NV = _N // _VL
_UNROLL = 8


def _sc_body(sim0_ref, sim1_ref, sim2_ref, mask0_ref, mask1_ref, mask2_ref,
             out_ref, rowbuf, maskbuf, ukeybuf, hist, ctot, outbuf,
             sem_sim, sem_mask):
    wid = lax.axis_index("s") * _NC + lax.axis_index("c")
    lanes = lax.iota(jnp.int32, _VL)
    lanes257 = lanes * 257
    ones_i32 = jnp.ones((_VL,), jnp.int32)
    ones_vec = ones_i32
    zeros_i32 = jnp.zeros((_VL,), jnp.int32)

    def clr(i, _):
        hist[pl.ds(i * _VL, _VL)] = zeros_i32
        return 0

    lax.fori_loop(0, 16 * 257 // _VL + 1, clr, 0)

    def do_stage(stage, sim_ref, mask_ref):
        base = wid * _RPW
        pltpu.async_copy(sim_ref.at[base], rowbuf.at[0], sem_sim.at[0])
        pltpu.async_copy(mask_ref.at[base], maskbuf.at[0], sem_mask.at[0])

        def do_row(j, carry):
            par = jnp.bitwise_and(j, 1)
            row = base + j

            @pl.when(j < _RPW - 1)
            def _prefetch():
                nxt = 1 - par
                pltpu.async_copy(sim_ref.at[row + 1], rowbuf.at[nxt],
                                 sem_sim.at[nxt])
                pltpu.async_copy(mask_ref.at[row + 1], maskbuf.at[nxt],
                                 sem_mask.at[nxt])

            pltpu.make_async_copy(sim_ref.at[row], rowbuf.at[par],
                                  sem_sim.at[par]).wait()
            pltpu.make_async_copy(mask_ref.at[row], maskbuf.at[par],
                                  sem_mask.at[par]).wait()

            @plsc.parallel_loop(0, _NV, 1, unroll=_UNROLL)
            def _build(i):
                x = rowbuf[par, pl.ds(i * _VL, _VL)]
                bits = plsc.bitcast(x, jnp.int32)
                ukey = jnp.where(bits < 0, ~bits, bits | _INT_MIN)
                ukey = jnp.where(x == 0.0, _INT_MIN, ukey)
                ukeybuf[pl.ds(i * _VL, _VL)] = ukey
                ubin = lax.shift_right_logical(ukey, 24)
                plsc.addupdate_scatter(hist, [lanes257 + ubin], ones_i32)

            v = jnp.int32(0)
            kk = jnp.int32(_K)

            for p in range(4):
                def chunk_body(c, csums):
                    acc = zeros_i32
                    for l in range(16):
                        sl = pl.ds(c * _VL + l * 257, _VL)
                        acc = acc + hist[sl]
                        hist[sl] = zeros_i32
                    ctot[pl.ds(c * _VL, _VL)] = acc
                    return jnp.where(lanes == c, jnp.sum(acc), csums)

                csums = lax.fori_loop(0, 16, chunk_body, zeros_i32)

                total = jnp.sum(csums)
                above_chunk = total - plsc.cumsum(csums)
                cstar = jnp.max(plsc.all_reduce_ffs(above_chunk < kk))
                above_cstar = jnp.sum(jnp.where(lanes == cstar, above_chunk, 0))
                bt = ctot[pl.ds(cstar * _VL, _VL)]
                insuf = jnp.sum(bt) - plsc.cumsum(bt)
                bl = jnp.max(plsc.all_reduce_ffs((above_cstar + insuf) < kk))
                above_b = above_cstar + jnp.sum(jnp.where(lanes == bl, insuf, 0))
                v = jnp.left_shift(v, 8) | (cstar * _VL + bl)
                kk = kk - above_b

                if p < 3:
                    shift = 24 - 8 * (p + 1)
                    vv = jnp.full((_VL,), v, jnp.int32)

                    @plsc.parallel_loop(0, _NV, 1, unroll=_UNROLL)
                    def _histpass(i):
                        ukey = ukeybuf[pl.ds(i * _VL, _VL)]
                        binfull = lax.shift_right_logical(ukey, shift)
                        hi = lax.shift_right_logical(binfull, 8)
                        low = binfull & 255
                        plsc.addupdate_scatter(hist, [lanes257 + low],
                                               ones_i32, mask=hi == vv)

            tkey = jnp.full((_VL,), v ^ _INT_MIN, jnp.int32)

            zf = jnp.zeros((_VL,), jnp.float32)

            @plsc.parallel_loop(0, _NV, 1, unroll=_UNROLL, carry=(zf, zf))
            def accs(i, acc):
                a1, a2 = acc
                sk = ukeybuf[pl.ds(i * _VL, _VL)] ^ _INT_MIN
                m = maskbuf[par, pl.ds(i * _VL, _VL)]
                a1 = a1 + jnp.where(sk < tkey, 1.0 - 2.0 * m, 0.0)
                return a1, a2 + m * m

            return carry + jnp.sum(accs[0]) + jnp.sum(accs[1])

        s1 = lax.fori_loop(0, _RPW, do_row, jnp.float32(0.0))
        outbuf[...] = jnp.where(lanes == 0, s1, 0.0)
        pltpu.sync_copy(outbuf, out_ref.at[stage, wid])

    do_stage(0, sim0_ref, mask0_ref)
    do_stage(1, sim1_ref, mask1_ref)
    do_stage(2, sim2_ref, mask2_ref)


def _sc_call(sims, masks, interpret=False):
    mesh = plsc.VectorSubcoreMesh(core_axis_name="c", subcore_axis_name="s",
                                  num_cores=_NC, num_subcores=_NS)
    fn = pl.kernel(
        _sc_body,
        out_type=jax.ShapeDtypeStruct((_NSTAGE, _NW, _VL), jnp.float32),
        mesh=mesh,
        scratch_types=[
            pltpu.VMEM((2, _N), jnp.float32),
            pltpu.VMEM((2, _N), jnp.float32),
            pltpu.VMEM((_N,), jnp.int32),
            pltpu.VMEM((16 * 257 + _VL,), jnp.int32),
            pltpu.VMEM((256,), jnp.int32),
            pltpu.VMEM((_VL,), jnp.float32),
            pltpu.SemaphoreType.DMA((2,)),
            pltpu.SemaphoreType.DMA((2,)),
        ],
        compiler_params=pltpu.CompilerParams(needs_layout_passes=False),
        interpret=interpret,
    )
    return fn(*sims, *masks)


def _m2_body(m0_ref, m1_ref, m2_ref, out_ref):
    pid = pl.program_id(0)

    @pl.when(pid == 0)
    def _init():
        out_ref[...] = jnp.zeros((1, 1), jnp.float32)

    acc = jnp.float32(0.0)
    for ref in (m0_ref, m1_ref, m2_ref):
        m = ref[...]
        acc = acc + jnp.sum(m * m)
    out_ref[...] += acc.reshape(1, 1)


def _m2_call(masks):
    rows = 16
    return pl.pallas_call(
        _m2_body,
        grid=(_B // rows,),
        in_specs=[pl.BlockSpec((rows, _N), lambda i: (i, 0))] * _NSTAGE,
        out_specs=pl.BlockSpec((1, 1), lambda i: (0, 0)),
        out_shape=jax.ShapeDtypeStruct((1, 1), jnp.float32),
    )(*masks)


@jax.jit
def kernel(pred_mask_0, pred_mask_1, pred_mask_2,
           token_attn_sim_0, token_attn_sim_1, token_attn_sim_2):
    sims = [t[:, :, 1] for t in
            (token_attn_sim_0, token_attn_sim_1, token_attn_sim_2)]
    masks = [pred_mask_0, pred_mask_1, pred_mask_2]
    s1 = _sc_call(sims, masks)
    total = jnp.sum(s1[:, :, 0])
    return _ALPHA * total / jnp.float32(_NSTAGE * _B * _N)


# speedup vs baseline: 1.0149x; 1.0149x over previous
import jax
import jax.numpy as jnp
import numpy as np
from jax import lax
from jax.experimental import pallas as pl
from jax.experimental.pallas import tpu as pltpu
from jax.experimental.pallas import tpu_sc as plsc

_B = 128
_N = 8192
_K = 2457
_ALPHA = 2.0
_NSTAGE = 3
_INT_MIN = np.int32(-(2 ** 31))

_NC = 2
_NS = 16
_NW = _NC * _NS
_RPW = _B // _NW
_VL = 16
_NV = _N // _VL
_UNROLL = 8


def _sc_body(sim0_ref, sim1_ref, sim2_ref, mask0_ref, mask1_ref, mask2_ref,
             out_ref, rowbuf, maskbuf, ukeybuf, hist, ctot, outbuf,
             sem_sim, sem_mask):
    wid = lax.axis_index("s") * _NC + lax.axis_index("c")
    lanes = lax.iota(jnp.int32, _VL)
    lanes257 = lanes * 257
    ones_i32 = jnp.ones((_VL,), jnp.int32)
    ones_vec = ones_i32
    zeros_i32 = jnp.zeros((_VL,), jnp.int32)

    def clr(i, _):
        hist[pl.ds(i * _VL, _VL)] = zeros_i32
        return 0

    lax.fori_loop(0, 16 * 257 // _VL + 1, clr, 0)

    def do_stage(stage, sim_ref, mask_ref):
        base = wid * _RPW
        pltpu.async_copy(sim_ref.at[base], rowbuf.at[0], sem_sim.at[0])
        pltpu.async_copy(mask_ref.at[base], maskbuf.at[0], sem_mask.at[0])

        def do_row(j, carry):
            par = jnp.bitwise_and(j, 1)
            row = base + j

            @pl.when(j < _RPW - 1)
            def _prefetch():
                nxt = 1 - par
                pltpu.async_copy(sim_ref.at[row + 1], rowbuf.at[nxt],
                                 sem_sim.at[nxt])
                pltpu.async_copy(mask_ref.at[row + 1], maskbuf.at[nxt],
                                 sem_mask.at[nxt])

            pltpu.make_async_copy(sim_ref.at[row], rowbuf.at[par],
                                  sem_sim.at[par]).wait()
            pltpu.make_async_copy(mask_ref.at[row], maskbuf.at[par],
                                  sem_mask.at[par]).wait()

            @plsc.parallel_loop(0, _NV, 1, unroll=_UNROLL)
            def _build(i):
                x = rowbuf[par, pl.ds(i * _VL, _VL)]
                bits = plsc.bitcast(x, jnp.int32)
                ukey = jnp.where(bits < 0, ~bits, bits | _INT_MIN)
                ukey = jnp.where(x == 0.0, _INT_MIN, ukey)
                ukeybuf[pl.ds(i * _VL, _VL)] = ukey
                ubin = lax.shift_right_logical(ukey, 24)
                plsc.addupdate_scatter(hist, [lanes257 + ubin], ones_i32)

            v = jnp.int32(0)
            kk = jnp.int32(_K)

            for p in range(4):
                def chunk_body(c, csums):
                    acc = zeros_i32
                    for l in range(16):
                        sl = pl.ds(c * _VL + l * 257, _VL)
                        acc = acc + hist[sl]
                        hist[sl] = zeros_i32
                    ctot[pl.ds(c * _VL, _VL)] = acc
                    return jnp.where(lanes == c, jnp.sum(acc), csums)

                csums = lax.fori_loop(0, 16, chunk_body, zeros_i32)

                total = jnp.sum(csums)
                above_chunk = total - plsc.cumsum(csums)
                cstar = jnp.max(plsc.all_reduce_ffs(above_chunk < kk))
                above_cstar = jnp.sum(jnp.where(lanes == cstar, above_chunk, 0))
                bt = ctot[pl.ds(cstar * _VL, _VL)]
                insuf = jnp.sum(bt) - plsc.cumsum(bt)
                bl = jnp.max(plsc.all_reduce_ffs((above_cstar + insuf) < kk))
                above_b = above_cstar + jnp.sum(jnp.where(lanes == bl, insuf, 0))
                v = jnp.left_shift(v, 8) | (cstar * _VL + bl)
                kk = kk - above_b

                if p < 3:
                    shift = 24 - 8 * (p + 1)
                    vv = jnp.full((_VL,), v, jnp.int32)

                    @plsc.parallel_loop(0, _NV, 1, unroll=_UNROLL)
                    def _histpass(i):
                        ukey = ukeybuf[pl.ds(i * _VL, _VL)]
                        binfull = lax.shift_right_logical(ukey, shift)
                        hi = lax.shift_right_logical(binfull, 8)
                        low = binfull & 255
                        plsc.addupdate_scatter(hist, [lanes257 + low],
                                               ones_i32, mask=hi == vv)

            tkey = jnp.full((_VL,), v ^ _INT_MIN, jnp.int32)

            @plsc.parallel_loop(0, _NV, 1, unroll=_UNROLL,
                                carry=jnp.zeros((_VL,), jnp.float32))
            def accf(i, acc):
                sk = ukeybuf[pl.ds(i * _VL, _VL)] ^ _INT_MIN
                m = maskbuf[par, pl.ds(i * _VL, _VL)]
                return acc + jnp.where(sk < tkey, 1.0 - 2.0 * m, 0.0)

            return carry + jnp.sum(accf)

        s1 = lax.fori_loop(0, _RPW, do_row, jnp.float32(0.0))
        outbuf[...] = jnp.where(lanes == 0, s1, 0.0)
        pltpu.sync_copy(outbuf, out_ref.at[stage, wid])

    do_stage(0, sim0_ref, mask0_ref)
    do_stage(1, sim1_ref, mask1_ref)
    do_stage(2, sim2_ref, mask2_ref)


def _sc_call(sims, masks, interpret=False):
    mesh = plsc.VectorSubcoreMesh(core_axis_name="c", subcore_axis_name="s",
                                  num_cores=_NC, num_subcores=_NS)
    fn = pl.kernel(
        _sc_body,
        out_type=jax.ShapeDtypeStruct((_NSTAGE, _NW, _VL), jnp.float32),
        mesh=mesh,
        scratch_types=[
            pltpu.VMEM((2, _N), jnp.float32),
            pltpu.VMEM((2, _N), jnp.float32),
            pltpu.VMEM((_N,), jnp.int32),
            pltpu.VMEM((16 * 257 + _VL,), jnp.int32),
            pltpu.VMEM((256,), jnp.int32),
            pltpu.VMEM((_VL,), jnp.float32),
            pltpu.SemaphoreType.DMA((2,)),
            pltpu.SemaphoreType.DMA((2,)),
        ],
        compiler_params=pltpu.CompilerParams(needs_layout_passes=False),
        interpret=interpret,
    )
    return fn(*sims, *masks)


def _m2_body(m0_ref, m1_ref, m2_ref, out_ref):
    pid = pl.program_id(0)

    @pl.when(pid == 0)
    def _init():
        out_ref[...] = jnp.zeros((1, 1), jnp.float32)

    acc = jnp.float32(0.0)
    for ref in (m0_ref, m1_ref, m2_ref):
        m = ref[...]
        acc = acc + jnp.sum(m * m)
    out_ref[...] += acc.reshape(1, 1)


def _m2_call(masks):
    rows = 16
    return pl.pallas_call(
        _m2_body,
        grid=(_B // rows,),
        in_specs=[pl.BlockSpec((rows, _N), lambda i: (i, 0))] * _NSTAGE,
        out_specs=pl.BlockSpec((1, 1), lambda i: (0, 0)),
        out_shape=jax.ShapeDtypeStruct((1, 1), jnp.float32),
    )(*masks)


@jax.jit
def kernel(pred_mask_0, pred_mask_1, pred_mask_2,
           token_attn_sim_0, token_attn_sim_1, token_attn_sim_2):
    sims = [t[:, :, 1] for t in
            (token_attn_sim_0, token_attn_sim_1, token_attn_sim_2)]
    masks = [pred_mask_0, pred_mask_1, pred_mask_2]
    s1 = _sc_call(sims, masks)
    m2 = _m2_call(masks)
    total = jnp.sum(s1[:, :, 0]) + m2[0, 0]
    return _ALPHA * total / jnp.float32(_NSTAGE * _B * _N)
